# Initial kernel scaffold; baseline (speedup 1.0000x reference)
import jax
import jax.numpy as jnp
from jax.experimental import pallas as pl


def kernel(x, edge_index, W1, b1, W2, b2):
    raise NotImplementedError("write your pallas kernel here")


# speedup vs baseline: 10.5292x; 10.5292x over previous
import functools

import jax
import jax.numpy as jnp
from jax import lax
from jax.experimental import pallas as pl
from jax.experimental.pallas import tpu as pltpu
from jax.experimental.pallas import tpu_sc as plsc

_N = 10000
_D = 128
_DH = 256
_DHALF = _D // 2

_NC = 2
_NS = 16
_NW = _NC * _NS
_NCH = 250
_C = 40
_RPS = 640
_NR = _NS * _RPS

_mesh = plsc.VectorSubcoreMesh(core_axis_name="c", subcore_axis_name="s")


@functools.partial(
    pl.kernel,
    out_type=jax.ShapeDtypeStruct((_NC * _NR,), jnp.float32),
    mesh=_mesh,
    scratch_types=[
        pltpu.VMEM((_NCH, _C), jnp.int32),
        pltpu.VMEM((48,), jnp.float32),
        pltpu.VMEM((_RPS,), jnp.float32),
        pltpu.VMEM_SHARED((_NR,), jnp.float32),
    ],
)
def _deg_sc(dst_hbm, out_hbm, dst_v, ones_v, z_v, acc_sh):
    c = lax.axis_index("c")
    s = lax.axis_index("s")
    wid = c * _NS + s
    pltpu.sync_copy(dst_hbm.at[wid], dst_v)
    for i in range(3):
        ones_v[pl.ds(i * 16, 16)] = jnp.ones((16,), jnp.float32)
    for i in range(_RPS // 16):
        z_v[pl.ds(i * 16, 16)] = jnp.zeros((16,), jnp.float32)
    pltpu.sync_copy(z_v, acc_sh.at[pl.ds(s * _RPS, _RPS)])
    plsc.subcore_barrier()

    def body(j, carry):
        pltpu.sync_copy(ones_v.at[pl.ds(0, _C)], acc_sh.at[dst_v.at[j]], add=True)
        return carry

    lax.fori_loop(0, _NCH, body, 0)
    plsc.subcore_barrier()
    pltpu.sync_copy(acc_sh.at[pl.ds(s * _RPS, _RPS)],
                    out_hbm.at[pl.ds(c * _NR + s * _RPS, _RPS)])


@functools.partial(
    pl.kernel,
    out_type=jax.ShapeDtypeStruct((_NC, 2, _NR, _DHALF), jnp.float32),
    mesh=_mesh,
    scratch_types=[
        pltpu.VMEM((_NCH, _C), jnp.int32),
        pltpu.VMEM((_NCH, _C), jnp.int32),
        pltpu.VMEM((_C, _DHALF), jnp.float32),
        pltpu.VMEM_SHARED((_NR, _DHALF), jnp.float32),
        pltpu.SemaphoreType.DMA,
    ],
    compiler_params=pltpu.CompilerParams(use_tc_tiling_on_sc=False),
)
def _prop_sc(tab_lo, tab_hi, src_hbm, dst_hbm, out_hbm,
             src_v, dst_v, rows_v, acc_sh, sem):
    c = lax.axis_index("c")
    s = lax.axis_index("s")
    wid = c * _NS + s
    pltpu.sync_copy(src_hbm.at[wid], src_v)
    pltpu.sync_copy(dst_hbm.at[wid], dst_v)
    rs = s * _RPS
    for h, tab in enumerate((tab_lo, tab_hi)):
        pltpu.sync_copy(tab.at[pl.ds(rs, _RPS)], acc_sh.at[pl.ds(rs, _RPS)])
        plsc.subcore_barrier()

        def body(j, carry):
            pltpu.async_copy(tab.at[src_v.at[j]], rows_v, sem).wait()
            pltpu.sync_copy(rows_v, acc_sh.at[dst_v.at[j]], add=True)
            return carry

        lax.fori_loop(0, _NCH, body, 0)
        plsc.subcore_barrier()
        pltpu.sync_copy(acc_sh.at[pl.ds(rs, _RPS)], out_hbm.at[c, h, pl.ds(rs, _RPS)])
        plsc.subcore_barrier()


_BN = 400
_GRID = _N // _BN


def _dinv_of(deg_ref):
    d = deg_ref[:, 0:1] + deg_ref[:, 1:2] + 1.0
    return lax.rsqrt(d)


def _scale_body(deg_ref, x_ref, lo_ref, hi_ref):
    dinv = _dinv_of(deg_ref)
    lo_ref[...] = x_ref[:, 0:_DHALF] * dinv
    hi_ref[...] = x_ref[:, _DHALF:_D] * dinv


def _mid_body(deg_ref, g_ref, xlo_ref, xhi_ref, w1_ref, b1_ref, w2_ref,
              lo_ref, hi_ref):
    dinv = _dinv_of(deg_ref)
    p_lo = (g_ref[0, 0] + g_ref[1, 0] - xlo_ref[...]) * dinv
    p_hi = (g_ref[0, 1] + g_ref[1, 1] - xhi_ref[...]) * dinv
    h = jnp.maximum(
        jnp.dot(p_lo, w1_ref[0:_DHALF], preferred_element_type=jnp.float32)
        + jnp.dot(p_hi, w1_ref[_DHALF:_D], preferred_element_type=jnp.float32)
        + b1_ref[...], 0.0)
    lo_ref[...] = jnp.dot(h, w2_ref[:, 0:_DHALF],
                          preferred_element_type=jnp.float32) * dinv
    hi_ref[...] = jnp.dot(h, w2_ref[:, _DHALF:_D],
                          preferred_element_type=jnp.float32) * dinv


def _final_body(deg_ref, q_ref, hlo_ref, hhi_ref, b2_ref, o_ref):
    dinv = _dinv_of(deg_ref)
    o_lo = (q_ref[0, 0] + q_ref[1, 0] - hlo_ref[...]) * dinv + b2_ref[:, 0:_DHALF]
    o_hi = (q_ref[0, 1] + q_ref[1, 1] - hhi_ref[...]) * dinv + b2_ref[:, _DHALF:_D]
    m = jnp.maximum(jnp.max(o_lo, axis=1, keepdims=True),
                    jnp.max(o_hi, axis=1, keepdims=True))
    lse = jnp.log(jnp.sum(jnp.exp(o_lo - m), axis=1, keepdims=True)
                  + jnp.sum(jnp.exp(o_hi - m), axis=1, keepdims=True))
    o_ref[:, 0:_DHALF] = o_lo - m - lse
    o_ref[:, _DHALF:_D] = o_hi - m - lse


_deg_spec = pl.BlockSpec((_BN, _NC), lambda i: (i, 0))
_row_spec = pl.BlockSpec((_BN, _D), lambda i: (i, 0))
_half_spec = pl.BlockSpec((_BN, _DHALF), lambda i: (i, 0))
_par_spec = pl.BlockSpec((_NC, 2, _BN, _DHALF), lambda i: (0, 0, i, 0))
_half_out = jax.ShapeDtypeStruct((_NR, _DHALF), jnp.float32)

_scale_tc = pl.pallas_call(
    _scale_body,
    grid=(_GRID,),
    in_specs=[_deg_spec, _row_spec],
    out_specs=(_half_spec, _half_spec),
    out_shape=(_half_out, _half_out),
)

_mid_tc = pl.pallas_call(
    _mid_body,
    grid=(_GRID,),
    in_specs=[
        _deg_spec,
        _par_spec,
        _half_spec,
        _half_spec,
        pl.BlockSpec((_D, _DH), lambda i: (0, 0)),
        pl.BlockSpec((1, _DH), lambda i: (0, 0)),
        pl.BlockSpec((_DH, _D), lambda i: (0, 0)),
    ],
    out_specs=(_half_spec, _half_spec),
    out_shape=(_half_out, _half_out),
)

_final_tc = pl.pallas_call(
    _final_body,
    grid=(_GRID,),
    in_specs=[_deg_spec, _par_spec, _half_spec, _half_spec,
              pl.BlockSpec((1, _D), lambda i: (0, 0))],
    out_specs=_row_spec,
    out_shape=jax.ShapeDtypeStruct((_N, _D), jnp.float32),
)


def kernel(x, edge_index, W1, b1, W2, b2):
    src = edge_index[0].reshape(_NW, _NCH, _C)
    dst = edge_index[1].reshape(_NW, _NCH, _C)
    deg = _deg_sc(dst).reshape(_NC, _NR)
    deg_nt = deg.T
    xs_lo, xs_hi = _scale_tc(deg_nt, x)
    g = _prop_sc(xs_lo, xs_hi, src, dst)
    h2s_lo, h2s_hi = _mid_tc(deg_nt, g, xs_lo, xs_hi,
                             W1, b1.reshape(1, _DH), W2)
    q = _prop_sc(h2s_lo, h2s_hi, src, dst)
    return _final_tc(deg_nt, q, h2s_lo, h2s_hi, b2.reshape(1, _D))


# speedup vs baseline: 29.8078x; 2.8310x over previous
import functools

import jax
import jax.numpy as jnp
from jax import lax
from jax.experimental import pallas as pl
from jax.experimental.pallas import tpu as pltpu
from jax.experimental.pallas import tpu_sc as plsc

_N = 10000
_D = 128
_DH = 256
_DHALF = _D // 2

_NC = 2
_NS = 16
_NW = _NC * _NS
_NCH = 125
_C = 80
_NB = 5
_RPS = 640
_NR = _NS * _RPS

_mesh = plsc.VectorSubcoreMesh(core_axis_name="c", subcore_axis_name="s")


@functools.partial(
    pl.kernel,
    out_type=jax.ShapeDtypeStruct((_NC * _NR,), jnp.float32),
    mesh=_mesh,
    scratch_types=[
        pltpu.VMEM((_NCH, _C), jnp.int32),
        pltpu.VMEM((_C,), jnp.float32),
        pltpu.VMEM((_RPS,), jnp.float32),
        pltpu.VMEM_SHARED((_NR,), jnp.float32),
    ],
)
def _deg_sc(dst_hbm, out_hbm, dst_v, ones_v, z_v, acc_sh):
    c = lax.axis_index("c")
    s = lax.axis_index("s")
    wid = c * _NS + s
    pltpu.sync_copy(dst_hbm.at[wid], dst_v)
    for i in range(_C // 16):
        ones_v[pl.ds(i * 16, 16)] = jnp.ones((16,), jnp.float32)
    for i in range(_RPS // 16):
        z_v[pl.ds(i * 16, 16)] = jnp.zeros((16,), jnp.float32)
    pltpu.sync_copy(z_v, acc_sh.at[pl.ds(s * _RPS, _RPS)])
    plsc.subcore_barrier()

    def body(j, carry):
        pltpu.sync_copy(ones_v.at[pl.ds(0, _C)], acc_sh.at[dst_v.at[j]], add=True)
        return carry

    lax.fori_loop(0, _NCH, body, 0)
    plsc.subcore_barrier()
    pltpu.sync_copy(acc_sh.at[pl.ds(s * _RPS, _RPS)],
                    out_hbm.at[pl.ds(c * _NR + s * _RPS, _RPS)])


@functools.partial(
    pl.kernel,
    out_type=jax.ShapeDtypeStruct((_NC, 2, _NR, _DHALF), jnp.float32),
    mesh=_mesh,
    scratch_types=[
        pltpu.VMEM((_NCH, _C), jnp.int32),
        pltpu.VMEM((_NCH, _C), jnp.int32),
        pltpu.VMEM((_NB, _C, _DHALF), jnp.float32),
        pltpu.VMEM_SHARED((_NR, _DHALF), jnp.float32),
    ] + [pltpu.SemaphoreType.DMA] * _NB,
    compiler_params=pltpu.CompilerParams(use_tc_tiling_on_sc=False),
)
def _prop_sc(tab_lo, tab_hi, src_hbm, dst_hbm, out_hbm,
             src_v, dst_v, rows_v, acc_sh, *sems):
    c = lax.axis_index("c")
    s = lax.axis_index("s")
    wid = c * _NS + s
    pltpu.sync_copy(src_hbm.at[wid], src_v)
    pltpu.sync_copy(dst_hbm.at[wid], dst_v)
    rs = s * _RPS
    for h, tab in enumerate((tab_lo, tab_hi)):
        pltpu.sync_copy(tab.at[pl.ds(rs, _RPS)], acc_sh.at[pl.ds(rs, _RPS)])
        plsc.subcore_barrier()

        for b in range(_NB):
            pltpu.async_copy(tab.at[src_v.at[b]], rows_v.at[b], sems[b])

        def round_body(r, carry):
            for b in range(_NB):
                g = r * _NB + b
                pltpu.make_async_copy(tab.at[src_v.at[g]], rows_v.at[b],
                                      sems[b]).wait()
                pltpu.sync_copy(rows_v.at[b], acc_sh.at[dst_v.at[g]], add=True)

                @pl.when(r < _NCH // _NB - 1)
                def _():
                    pltpu.async_copy(tab.at[src_v.at[g + _NB]], rows_v.at[b],
                                     sems[b])
            return carry

        lax.fori_loop(0, _NCH // _NB, round_body, 0)
        plsc.subcore_barrier()
        pltpu.sync_copy(acc_sh.at[pl.ds(rs, _RPS)], out_hbm.at[c, h, pl.ds(rs, _RPS)])
        plsc.subcore_barrier()


_BN = 400
_GRID = _N // _BN


def _dinv_of(deg_ref):
    d = deg_ref[:, 0:1] + deg_ref[:, 1:2] + 1.0
    return lax.rsqrt(d)


def _scale_body(deg_ref, x_ref, lo_ref, hi_ref):
    dinv = _dinv_of(deg_ref)
    lo_ref[...] = x_ref[:, 0:_DHALF] * dinv
    hi_ref[...] = x_ref[:, _DHALF:_D] * dinv


def _mid_body(deg_ref, g_ref, xlo_ref, xhi_ref, w1_ref, b1_ref, w2_ref,
              lo_ref, hi_ref):
    dinv = _dinv_of(deg_ref)
    p_lo = (g_ref[0, 0] + g_ref[1, 0] - xlo_ref[...]) * dinv
    p_hi = (g_ref[0, 1] + g_ref[1, 1] - xhi_ref[...]) * dinv
    h = jnp.maximum(
        jnp.dot(p_lo, w1_ref[0:_DHALF], preferred_element_type=jnp.float32)
        + jnp.dot(p_hi, w1_ref[_DHALF:_D], preferred_element_type=jnp.float32)
        + b1_ref[...], 0.0)
    lo_ref[...] = jnp.dot(h, w2_ref[:, 0:_DHALF],
                          preferred_element_type=jnp.float32) * dinv
    hi_ref[...] = jnp.dot(h, w2_ref[:, _DHALF:_D],
                          preferred_element_type=jnp.float32) * dinv


def _final_body(deg_ref, q_ref, hlo_ref, hhi_ref, b2_ref, o_ref):
    dinv = _dinv_of(deg_ref)
    o_lo = (q_ref[0, 0] + q_ref[1, 0] - hlo_ref[...]) * dinv + b2_ref[:, 0:_DHALF]
    o_hi = (q_ref[0, 1] + q_ref[1, 1] - hhi_ref[...]) * dinv + b2_ref[:, _DHALF:_D]
    m = jnp.maximum(jnp.max(o_lo, axis=1, keepdims=True),
                    jnp.max(o_hi, axis=1, keepdims=True))
    lse = jnp.log(jnp.sum(jnp.exp(o_lo - m), axis=1, keepdims=True)
                  + jnp.sum(jnp.exp(o_hi - m), axis=1, keepdims=True))
    o_ref[:, 0:_DHALF] = o_lo - m - lse
    o_ref[:, _DHALF:_D] = o_hi - m - lse


_deg_spec = pl.BlockSpec((_BN, _NC), lambda i: (i, 0))
_row_spec = pl.BlockSpec((_BN, _D), lambda i: (i, 0))
_half_spec = pl.BlockSpec((_BN, _DHALF), lambda i: (i, 0))
_par_spec = pl.BlockSpec((_NC, 2, _BN, _DHALF), lambda i: (0, 0, i, 0))
_half_out = jax.ShapeDtypeStruct((_NR, _DHALF), jnp.float32)

_scale_tc = pl.pallas_call(
    _scale_body,
    grid=(_GRID,),
    in_specs=[_deg_spec, _row_spec],
    out_specs=(_half_spec, _half_spec),
    out_shape=(_half_out, _half_out),
)

_mid_tc = pl.pallas_call(
    _mid_body,
    grid=(_GRID,),
    in_specs=[
        _deg_spec,
        _par_spec,
        _half_spec,
        _half_spec,
        pl.BlockSpec((_D, _DH), lambda i: (0, 0)),
        pl.BlockSpec((1, _DH), lambda i: (0, 0)),
        pl.BlockSpec((_DH, _D), lambda i: (0, 0)),
    ],
    out_specs=(_half_spec, _half_spec),
    out_shape=(_half_out, _half_out),
)

_final_tc = pl.pallas_call(
    _final_body,
    grid=(_GRID,),
    in_specs=[_deg_spec, _par_spec, _half_spec, _half_spec,
              pl.BlockSpec((1, _D), lambda i: (0, 0))],
    out_specs=_row_spec,
    out_shape=jax.ShapeDtypeStruct((_N, _D), jnp.float32),
)


def kernel(x, edge_index, W1, b1, W2, b2):
    src = edge_index[0].reshape(_NW, _NCH, _C)
    dst = edge_index[1].reshape(_NW, _NCH, _C)
    deg = _deg_sc(dst).reshape(_NC, _NR)
    deg_nt = deg.T
    xs_lo, xs_hi = _scale_tc(deg_nt, x)
    g = _prop_sc(xs_lo, xs_hi, src, dst)
    h2s_lo, h2s_hi = _mid_tc(deg_nt, g, xs_lo, xs_hi,
                             W1, b1.reshape(1, _DH), W2)
    q = _prop_sc(h2s_lo, h2s_hi, src, dst)
    return _final_tc(deg_nt, q, h2s_lo, h2s_hi, b2.reshape(1, _D))


# speedup vs baseline: 30.2878x; 1.0161x over previous
import functools

import jax
import jax.numpy as jnp
from jax import lax
from jax.experimental import pallas as pl
from jax.experimental.pallas import tpu as pltpu
from jax.experimental.pallas import tpu_sc as plsc

_N = 10000
_D = 128
_DH = 256
_DHALF = _D // 2

_NC = 2
_NS = 16
_NW = _NC * _NS
_NCH = 125
_C = 80
_NB = 5
_RPS = 640
_NR = _NS * _RPS

_mesh = plsc.VectorSubcoreMesh(core_axis_name="c", subcore_axis_name="s")


@functools.partial(
    pl.kernel,
    out_type=jax.ShapeDtypeStruct((_NC * _NR,), jnp.float32),
    mesh=_mesh,
    scratch_types=[
        pltpu.VMEM((_NCH, _C), jnp.int32),
        pltpu.VMEM((_C,), jnp.float32),
        pltpu.VMEM((_RPS,), jnp.float32),
        pltpu.VMEM_SHARED((_NR,), jnp.float32),
    ],
)
def _deg_sc(dst_hbm, out_hbm, dst_v, ones_v, z_v, acc_sh):
    c = lax.axis_index("c")
    s = lax.axis_index("s")
    wid = c * _NS + s
    pltpu.sync_copy(dst_hbm.at[wid], dst_v)
    for i in range(_C // 16):
        ones_v[pl.ds(i * 16, 16)] = jnp.ones((16,), jnp.float32)
    for i in range(_RPS // 16):
        z_v[pl.ds(i * 16, 16)] = jnp.zeros((16,), jnp.float32)
    pltpu.sync_copy(z_v, acc_sh.at[pl.ds(s * _RPS, _RPS)])
    plsc.subcore_barrier()

    def body(j, carry):
        pltpu.sync_copy(ones_v.at[pl.ds(0, _C)], acc_sh.at[dst_v.at[j]], add=True)
        return carry

    lax.fori_loop(0, _NCH, body, 0)
    plsc.subcore_barrier()
    pltpu.sync_copy(acc_sh.at[pl.ds(s * _RPS, _RPS)],
                    out_hbm.at[pl.ds(c * _NR + s * _RPS, _RPS)])


@functools.partial(
    pl.kernel,
    out_type=jax.ShapeDtypeStruct((_NC, 2, _NR, _DHALF), jnp.float32),
    mesh=_mesh,
    scratch_types=[
        pltpu.VMEM((_NCH, _C), jnp.int32),
        pltpu.VMEM((_NCH, _C), jnp.int32),
        pltpu.VMEM((_NB, _C, _DHALF), jnp.float32),
        pltpu.VMEM_SHARED((_NR, _DHALF), jnp.float32),
    ] + [pltpu.SemaphoreType.DMA] * (2 * _NB),
    compiler_params=pltpu.CompilerParams(use_tc_tiling_on_sc=False),
)
def _prop_sc(tab_lo, tab_hi, src_hbm, dst_hbm, out_hbm,
             src_v, dst_v, rows_v, acc_sh, *sems):
    c = lax.axis_index("c")
    s = lax.axis_index("s")
    wid = c * _NS + s
    pltpu.sync_copy(src_hbm.at[wid], src_v)
    pltpu.sync_copy(dst_hbm.at[wid], dst_v)
    rs = s * _RPS
    for h, tab in enumerate((tab_lo, tab_hi)):
        pltpu.sync_copy(tab.at[pl.ds(rs, _RPS)], acc_sh.at[pl.ds(rs, _RPS)])
        plsc.subcore_barrier()

        gsem = sems[:_NB]
        ssem = sems[_NB:]
        for b in range(_NB):
            pltpu.async_copy(tab.at[src_v.at[b]], rows_v.at[b], gsem[b])

        def round_body(r, carry):
            for b in range(_NB):
                g = r * _NB + b
                pltpu.make_async_copy(tab.at[src_v.at[g]], rows_v.at[b],
                                      gsem[b]).wait()
                pltpu.async_copy(rows_v.at[b], acc_sh.at[dst_v.at[g]],
                                 ssem[b], add=True)
                b2 = (b + _NB - 2) % _NB
                gg = g + _NB - 2

                @pl.when((gg >= _NB) & (gg < _NCH))
                def _():
                    pltpu.make_async_copy(rows_v.at[b2],
                                          acc_sh.at[dst_v.at[gg - _NB]],
                                          ssem[b2]).wait()
                    pltpu.async_copy(tab.at[src_v.at[gg]], rows_v.at[b2],
                                     gsem[b2])
            return carry

        lax.fori_loop(0, _NCH // _NB, round_body, 0)
        for b in range(_NB):
            pltpu.make_async_copy(rows_v.at[b],
                                  acc_sh.at[dst_v.at[_NCH - _NB + b]],
                                  ssem[b]).wait()
        plsc.subcore_barrier()
        pltpu.sync_copy(acc_sh.at[pl.ds(rs, _RPS)], out_hbm.at[c, h, pl.ds(rs, _RPS)])
        plsc.subcore_barrier()


_BN = 2000
_GRID = _N // _BN


def _dinv_of(deg_ref):
    i = pl.program_id(0)
    blk = deg_ref[pl.ds(i * _BN, _BN), :]
    d = blk[:, 0:1] + blk[:, 1:2] + 1.0
    return lax.rsqrt(d)


def _scale_body(deg_ref, x_ref, lo_ref, hi_ref):
    dinv = _dinv_of(deg_ref)
    lo_ref[...] = x_ref[:, 0:_DHALF] * dinv
    hi_ref[...] = x_ref[:, _DHALF:_D] * dinv


def _mid_body(deg_ref, g_ref, xlo_ref, xhi_ref, w1_ref, b1_ref, w2_ref,
              lo_ref, hi_ref):
    dinv = _dinv_of(deg_ref)
    p_lo = (g_ref[0, 0] + g_ref[1, 0] - xlo_ref[...]) * dinv
    p_hi = (g_ref[0, 1] + g_ref[1, 1] - xhi_ref[...]) * dinv
    h = jnp.maximum(
        jnp.dot(p_lo, w1_ref[0:_DHALF], preferred_element_type=jnp.float32)
        + jnp.dot(p_hi, w1_ref[_DHALF:_D], preferred_element_type=jnp.float32)
        + b1_ref[...], 0.0)
    lo_ref[...] = jnp.dot(h, w2_ref[:, 0:_DHALF],
                          preferred_element_type=jnp.float32) * dinv
    hi_ref[...] = jnp.dot(h, w2_ref[:, _DHALF:_D],
                          preferred_element_type=jnp.float32) * dinv


def _final_body(deg_ref, q_ref, hlo_ref, hhi_ref, b2_ref, o_ref):
    dinv = _dinv_of(deg_ref)
    o_lo = (q_ref[0, 0] + q_ref[1, 0] - hlo_ref[...]) * dinv + b2_ref[:, 0:_DHALF]
    o_hi = (q_ref[0, 1] + q_ref[1, 1] - hhi_ref[...]) * dinv + b2_ref[:, _DHALF:_D]
    m = jnp.maximum(jnp.max(o_lo, axis=1, keepdims=True),
                    jnp.max(o_hi, axis=1, keepdims=True))
    lse = jnp.log(jnp.sum(jnp.exp(o_lo - m), axis=1, keepdims=True)
                  + jnp.sum(jnp.exp(o_hi - m), axis=1, keepdims=True))
    o_ref[:, 0:_DHALF] = o_lo - m - lse
    o_ref[:, _DHALF:_D] = o_hi - m - lse


_deg_spec = pl.BlockSpec((_NR, _NC), lambda i: (0, 0))
_row_spec = pl.BlockSpec((_BN, _D), lambda i: (i, 0))
_half_spec = pl.BlockSpec((_BN, _DHALF), lambda i: (i, 0))
_par_spec = pl.BlockSpec((_NC, 2, _BN, _DHALF), lambda i: (0, 0, i, 0))
_half_out = jax.ShapeDtypeStruct((_NR, _DHALF), jnp.float32)

_scale_tc = pl.pallas_call(
    _scale_body,
    grid=(_GRID,),
    in_specs=[_deg_spec, _row_spec],
    out_specs=(_half_spec, _half_spec),
    out_shape=(_half_out, _half_out),
)

_mid_tc = pl.pallas_call(
    _mid_body,
    grid=(_GRID,),
    in_specs=[
        _deg_spec,
        _par_spec,
        _half_spec,
        _half_spec,
        pl.BlockSpec((_D, _DH), lambda i: (0, 0)),
        pl.BlockSpec((1, _DH), lambda i: (0, 0)),
        pl.BlockSpec((_DH, _D), lambda i: (0, 0)),
    ],
    out_specs=(_half_spec, _half_spec),
    out_shape=(_half_out, _half_out),
)

_final_tc = pl.pallas_call(
    _final_body,
    grid=(_GRID,),
    in_specs=[_deg_spec, _par_spec, _half_spec, _half_spec,
              pl.BlockSpec((1, _D), lambda i: (0, 0))],
    out_specs=_row_spec,
    out_shape=jax.ShapeDtypeStruct((_N, _D), jnp.float32),
)


def kernel(x, edge_index, W1, b1, W2, b2):
    src = edge_index[0].reshape(_NW, _NCH, _C)
    dst = edge_index[1].reshape(_NW, _NCH, _C)
    deg = _deg_sc(dst).reshape(_NC, _NR)
    deg_nt = deg.T
    xs_lo, xs_hi = _scale_tc(deg_nt, x)
    g = _prop_sc(xs_lo, xs_hi, src, dst)
    h2s_lo, h2s_hi = _mid_tc(deg_nt, g, xs_lo, xs_hi,
                             W1, b1.reshape(1, _DH), W2)
    q = _prop_sc(h2s_lo, h2s_hi, src, dst)
    return _final_tc(deg_nt, q, h2s_lo, h2s_hi, b2.reshape(1, _D))


# speedup vs baseline: 32.4166x; 1.0703x over previous
import functools

import jax
import jax.numpy as jnp
from jax import lax
from jax.experimental import pallas as pl
from jax.experimental.pallas import tpu as pltpu
from jax.experimental.pallas import tpu_sc as plsc

_N = 10000
_D = 128
_DH = 256
_DHALF = _D // 2

_NC = 2
_NS = 16
_NW = _NC * _NS
_NCH = 125
_C = 80
_NB = 5
_RPS = 640
_NR = _NS * _RPS

_mesh = plsc.VectorSubcoreMesh(core_axis_name="c", subcore_axis_name="s")


@functools.partial(
    pl.kernel,
    out_type=jax.ShapeDtypeStruct((_NC * _NR,), jnp.float32),
    mesh=_mesh,
    scratch_types=[
        pltpu.VMEM((_NCH, _C), jnp.int32),
        pltpu.VMEM((_C,), jnp.float32),
        pltpu.VMEM((_RPS,), jnp.float32),
        pltpu.VMEM_SHARED((_NR,), jnp.float32),
    ],
)
def _deg_sc(dst_hbm, out_hbm, dst_v, ones_v, z_v, acc_sh):
    c = lax.axis_index("c")
    s = lax.axis_index("s")
    wid = c * _NS + s
    pltpu.sync_copy(dst_hbm.at[wid], dst_v)
    for i in range(_C // 16):
        ones_v[pl.ds(i * 16, 16)] = jnp.ones((16,), jnp.float32)
    for i in range(_RPS // 16):
        z_v[pl.ds(i * 16, 16)] = jnp.zeros((16,), jnp.float32)
    pltpu.sync_copy(z_v, acc_sh.at[pl.ds(s * _RPS, _RPS)])
    plsc.subcore_barrier()

    def body(j, carry):
        pltpu.sync_copy(ones_v.at[pl.ds(0, _C)], acc_sh.at[dst_v.at[j]], add=True)
        return carry

    lax.fori_loop(0, _NCH, body, 0)
    plsc.subcore_barrier()
    pltpu.sync_copy(acc_sh.at[pl.ds(s * _RPS, _RPS)],
                    out_hbm.at[pl.ds(c * _NR + s * _RPS, _RPS)])


@functools.partial(
    pl.kernel,
    out_type=jax.ShapeDtypeStruct((_NC, 2, _NR, _DHALF), jnp.float32),
    mesh=_mesh,
    scratch_types=[
        pltpu.VMEM((_NCH, _C), jnp.int32),
        pltpu.VMEM((_NCH, _C), jnp.int32),
        pltpu.VMEM((_NB, _C, _DHALF), jnp.float32),
        pltpu.VMEM_SHARED((_NR, _DHALF), jnp.float32),
    ] + [pltpu.SemaphoreType.DMA] * (2 * _NB),
    compiler_params=pltpu.CompilerParams(use_tc_tiling_on_sc=False),
)
def _prop_sc(tab_lo, tab_hi, src_hbm, dst_hbm, out_hbm,
             src_v, dst_v, rows_v, acc_sh, *sems):
    c = lax.axis_index("c")
    s = lax.axis_index("s")
    wid = c * _NS + s
    pltpu.sync_copy(src_hbm.at[wid], src_v)
    pltpu.sync_copy(dst_hbm.at[wid], dst_v)
    rs = s * _RPS
    for h, tab in enumerate((tab_lo, tab_hi)):
        pltpu.sync_copy(tab.at[pl.ds(rs, _RPS)], acc_sh.at[pl.ds(rs, _RPS)])
        plsc.subcore_barrier()

        for b in range(_NB):
            pltpu.async_copy(tab.at[src_v.at[b]], rows_v.at[b], sems[b])

        def round_body(r, carry):
            for b in range(_NB):
                g = r * _NB + b
                pltpu.make_async_copy(tab.at[src_v.at[g]], rows_v.at[b],
                                      sems[b]).wait()
                pltpu.sync_copy(rows_v.at[b], acc_sh.at[dst_v.at[g]], add=True)

                @pl.when(r < _NCH // _NB - 1)
                def _():
                    pltpu.async_copy(tab.at[src_v.at[g + _NB]], rows_v.at[b],
                                     sems[b])
            return carry

        lax.fori_loop(0, _NCH // _NB, round_body, 0)
        plsc.subcore_barrier()
        pltpu.sync_copy(acc_sh.at[pl.ds(rs, _RPS)], out_hbm.at[c, h, pl.ds(rs, _RPS)])
        plsc.subcore_barrier()


_BN = 2000
_GRID = _N // _BN


def _dinv_of(deg_ref):
    i = pl.program_id(0)
    blk = deg_ref[pl.ds(i * _BN, _BN), :]
    d = blk[:, 0:1] + blk[:, 1:2] + 1.0
    return lax.rsqrt(d)


def _scale_body(deg_ref, x_ref, lo_ref, hi_ref):
    dinv = _dinv_of(deg_ref)
    lo_ref[...] = x_ref[:, 0:_DHALF] * dinv
    hi_ref[...] = x_ref[:, _DHALF:_D] * dinv


def _mid_body(deg_ref, g_ref, xlo_ref, xhi_ref, w1_ref, b1_ref, w2_ref,
              lo_ref, hi_ref):
    dinv = _dinv_of(deg_ref)
    p_lo = (g_ref[0, 0] + g_ref[1, 0] - xlo_ref[...]) * dinv
    p_hi = (g_ref[0, 1] + g_ref[1, 1] - xhi_ref[...]) * dinv
    h = jnp.maximum(
        jnp.dot(p_lo, w1_ref[0:_DHALF], preferred_element_type=jnp.float32)
        + jnp.dot(p_hi, w1_ref[_DHALF:_D], preferred_element_type=jnp.float32)
        + b1_ref[...], 0.0)
    lo_ref[...] = jnp.dot(h, w2_ref[:, 0:_DHALF],
                          preferred_element_type=jnp.float32) * dinv
    hi_ref[...] = jnp.dot(h, w2_ref[:, _DHALF:_D],
                          preferred_element_type=jnp.float32) * dinv


def _final_body(deg_ref, q_ref, hlo_ref, hhi_ref, b2_ref, o_ref):
    dinv = _dinv_of(deg_ref)
    o_lo = (q_ref[0, 0] + q_ref[1, 0] - hlo_ref[...]) * dinv + b2_ref[:, 0:_DHALF]
    o_hi = (q_ref[0, 1] + q_ref[1, 1] - hhi_ref[...]) * dinv + b2_ref[:, _DHALF:_D]
    m = jnp.maximum(jnp.max(o_lo, axis=1, keepdims=True),
                    jnp.max(o_hi, axis=1, keepdims=True))
    lse = jnp.log(jnp.sum(jnp.exp(o_lo - m), axis=1, keepdims=True)
                  + jnp.sum(jnp.exp(o_hi - m), axis=1, keepdims=True))
    o_ref[:, 0:_DHALF] = o_lo - m - lse
    o_ref[:, _DHALF:_D] = o_hi - m - lse


_deg_spec = pl.BlockSpec((_NR, _NC), lambda i: (0, 0))
_row_spec = pl.BlockSpec((_BN, _D), lambda i: (i, 0))
_half_spec = pl.BlockSpec((_BN, _DHALF), lambda i: (i, 0))
_par_spec = pl.BlockSpec((_NC, 2, _BN, _DHALF), lambda i: (0, 0, i, 0))
_half_out = jax.ShapeDtypeStruct((_NR, _DHALF), jnp.float32)

_scale_tc = pl.pallas_call(
    _scale_body,
    grid=(_GRID,),
    in_specs=[_deg_spec, _row_spec],
    out_specs=(_half_spec, _half_spec),
    out_shape=(_half_out, _half_out),
)

_mid_tc = pl.pallas_call(
    _mid_body,
    grid=(_GRID,),
    in_specs=[
        _deg_spec,
        _par_spec,
        _half_spec,
        _half_spec,
        pl.BlockSpec((_D, _DH), lambda i: (0, 0)),
        pl.BlockSpec((1, _DH), lambda i: (0, 0)),
        pl.BlockSpec((_DH, _D), lambda i: (0, 0)),
    ],
    out_specs=(_half_spec, _half_spec),
    out_shape=(_half_out, _half_out),
)

_final_tc = pl.pallas_call(
    _final_body,
    grid=(_GRID,),
    in_specs=[_deg_spec, _par_spec, _half_spec, _half_spec,
              pl.BlockSpec((1, _D), lambda i: (0, 0))],
    out_specs=_row_spec,
    out_shape=jax.ShapeDtypeStruct((_N, _D), jnp.float32),
)


def kernel(x, edge_index, W1, b1, W2, b2):
    src = edge_index[0].reshape(_NW, _NCH, _C)
    dst = edge_index[1].reshape(_NW, _NCH, _C)
    deg = _deg_sc(dst).reshape(_NC, _NR)
    deg_nt = deg.T
    xs_lo, xs_hi = _scale_tc(deg_nt, x)
    g = _prop_sc(xs_lo, xs_hi, src, dst)
    h2s_lo, h2s_hi = _mid_tc(deg_nt, g, xs_lo, xs_hi,
                             W1, b1.reshape(1, _DH), W2)
    q = _prop_sc(h2s_lo, h2s_hi, src, dst)
    return _final_tc(deg_nt, q, h2s_lo, h2s_hi, b2.reshape(1, _D))


# speedup vs baseline: 32.9748x; 1.0172x over previous
import functools

import jax
import jax.numpy as jnp
from jax import lax
from jax.experimental import pallas as pl
from jax.experimental.pallas import tpu as pltpu
from jax.experimental.pallas import tpu_sc as plsc

_N = 10000
_D = 128
_DH = 256
_DHALF = _D // 2

_NC = 2
_NS = 16
_NW = _NC * _NS
_NCH = 125
_C = 80
_NB = 5
_RPS = 640
_NR = _NS * _RPS

_mesh = plsc.VectorSubcoreMesh(core_axis_name="c", subcore_axis_name="s")


@functools.partial(
    pl.kernel,
    out_type=jax.ShapeDtypeStruct((_NC * _NR,), jnp.float32),
    mesh=_mesh,
    scratch_types=[
        pltpu.VMEM((_NCH, _C), jnp.int32),
        pltpu.VMEM((_C,), jnp.float32),
        pltpu.VMEM((_RPS,), jnp.float32),
        pltpu.VMEM_SHARED((_NR,), jnp.float32),
    ],
)
def _deg_sc(dst_hbm, out_hbm, dst_v, ones_v, z_v, acc_sh):
    c = lax.axis_index("c")
    s = lax.axis_index("s")
    wid = c * _NS + s
    pltpu.sync_copy(dst_hbm.at[wid], dst_v)
    for i in range(_C // 16):
        ones_v[pl.ds(i * 16, 16)] = jnp.ones((16,), jnp.float32)
    for i in range(_RPS // 16):
        z_v[pl.ds(i * 16, 16)] = jnp.zeros((16,), jnp.float32)
    pltpu.sync_copy(z_v, acc_sh.at[pl.ds(s * _RPS, _RPS)])
    plsc.subcore_barrier()

    def body(j, carry):
        pltpu.sync_copy(ones_v.at[pl.ds(0, _C)], acc_sh.at[dst_v.at[j]], add=True)
        return carry

    lax.fori_loop(0, _NCH, body, 0)
    plsc.subcore_barrier()
    pltpu.sync_copy(acc_sh.at[pl.ds(s * _RPS, _RPS)],
                    out_hbm.at[pl.ds(c * _NR + s * _RPS, _RPS)])


@functools.partial(
    pl.kernel,
    out_type=jax.ShapeDtypeStruct((_NC, _NR, _DHALF), jnp.float32),
    mesh=_mesh,
    scratch_types=[
        pltpu.VMEM((_NCH, _C), jnp.int32),
        pltpu.VMEM((_NCH, _C), jnp.int32),
        pltpu.VMEM((_NB, _C, _DHALF), jnp.float32),
        pltpu.VMEM_SHARED((_NR, _DHALF), jnp.float32),
    ] + [pltpu.SemaphoreType.DMA] * _NB,
    compiler_params=pltpu.CompilerParams(use_tc_tiling_on_sc=False),
)
def _prop_sc(tab, src_hbm, dst_hbm, out_hbm, src_v, dst_v, rows_v, acc_sh, *sems):
    c = lax.axis_index("c")
    s = lax.axis_index("s")
    wid = c * _NS + s
    pltpu.sync_copy(src_hbm.at[wid], src_v)
    pltpu.sync_copy(dst_hbm.at[wid], dst_v)
    rs = s * _RPS
    pltpu.sync_copy(tab.at[pl.ds(rs, _RPS)], acc_sh.at[pl.ds(rs, _RPS)])
    plsc.subcore_barrier()

    for b in range(_NB):
        pltpu.async_copy(tab.at[src_v.at[b]], rows_v.at[b], sems[b])

    def round_body(r, carry):
        for b in range(_NB):
            g = r * _NB + b
            pltpu.make_async_copy(tab.at[src_v.at[g]], rows_v.at[b],
                                  sems[b]).wait()
            pltpu.sync_copy(rows_v.at[b], acc_sh.at[dst_v.at[g]], add=True)

            @pl.when(r < _NCH // _NB - 1)
            def _():
                pltpu.async_copy(tab.at[src_v.at[g + _NB]], rows_v.at[b],
                                 sems[b])
        return carry

    lax.fori_loop(0, _NCH // _NB, round_body, 0)
    plsc.subcore_barrier()
    pltpu.sync_copy(acc_sh.at[pl.ds(rs, _RPS)], out_hbm.at[c, pl.ds(rs, _RPS)])


_BN = 2000
_GRID = _N // _BN


def _dinv_of(deg_ref):
    i = pl.program_id(0)
    blk = deg_ref[pl.ds(i * _BN, _BN), :]
    d = blk[:, 0:1] + blk[:, 1:2] + 1.0
    return lax.rsqrt(d)


def _scale_body(deg_ref, x_ref, lo_ref, hi_ref):
    dinv = _dinv_of(deg_ref)
    lo_ref[...] = x_ref[:, 0:_DHALF] * dinv
    hi_ref[...] = x_ref[:, _DHALF:_D] * dinv


def _mid_body(deg_ref, glo_ref, ghi_ref, xlo_ref, xhi_ref, w1_ref, b1_ref,
              w2_ref, lo_ref, hi_ref):
    dinv = _dinv_of(deg_ref)
    p_lo = (glo_ref[0] + glo_ref[1] - xlo_ref[...]) * dinv
    p_hi = (ghi_ref[0] + ghi_ref[1] - xhi_ref[...]) * dinv
    h = jnp.maximum(
        jnp.dot(p_lo, w1_ref[0:_DHALF], preferred_element_type=jnp.float32)
        + jnp.dot(p_hi, w1_ref[_DHALF:_D], preferred_element_type=jnp.float32)
        + b1_ref[...], 0.0)
    lo_ref[...] = jnp.dot(h, w2_ref[:, 0:_DHALF],
                          preferred_element_type=jnp.float32) * dinv
    hi_ref[...] = jnp.dot(h, w2_ref[:, _DHALF:_D],
                          preferred_element_type=jnp.float32) * dinv


def _final_body(deg_ref, qlo_ref, qhi_ref, hlo_ref, hhi_ref, b2_ref, o_ref):
    dinv = _dinv_of(deg_ref)
    o_lo = (qlo_ref[0] + qlo_ref[1] - hlo_ref[...]) * dinv + b2_ref[:, 0:_DHALF]
    o_hi = (qhi_ref[0] + qhi_ref[1] - hhi_ref[...]) * dinv + b2_ref[:, _DHALF:_D]
    m = jnp.maximum(jnp.max(o_lo, axis=1, keepdims=True),
                    jnp.max(o_hi, axis=1, keepdims=True))
    lse = jnp.log(jnp.sum(jnp.exp(o_lo - m), axis=1, keepdims=True)
                  + jnp.sum(jnp.exp(o_hi - m), axis=1, keepdims=True))
    o_ref[:, 0:_DHALF] = o_lo - m - lse
    o_ref[:, _DHALF:_D] = o_hi - m - lse


_deg_spec = pl.BlockSpec((_NR, _NC), lambda i: (0, 0))
_row_spec = pl.BlockSpec((_BN, _D), lambda i: (i, 0))
_half_spec = pl.BlockSpec((_BN, _DHALF), lambda i: (i, 0))
_par_spec = pl.BlockSpec((_NC, _BN, _DHALF), lambda i: (0, i, 0))
_half_out = jax.ShapeDtypeStruct((_NR, _DHALF), jnp.float32)

_scale_tc = pl.pallas_call(
    _scale_body,
    grid=(_GRID,),
    in_specs=[_deg_spec, _row_spec],
    out_specs=(_half_spec, _half_spec),
    out_shape=(_half_out, _half_out),
)

_mid_tc = pl.pallas_call(
    _mid_body,
    grid=(_GRID,),
    in_specs=[
        _deg_spec,
        _par_spec,
        _par_spec,
        _half_spec,
        _half_spec,
        pl.BlockSpec((_D, _DH), lambda i: (0, 0)),
        pl.BlockSpec((1, _DH), lambda i: (0, 0)),
        pl.BlockSpec((_DH, _D), lambda i: (0, 0)),
    ],
    out_specs=(_half_spec, _half_spec),
    out_shape=(_half_out, _half_out),
)

_final_tc = pl.pallas_call(
    _final_body,
    grid=(_GRID,),
    in_specs=[_deg_spec, _par_spec, _par_spec, _half_spec, _half_spec,
              pl.BlockSpec((1, _D), lambda i: (0, 0))],
    out_specs=_row_spec,
    out_shape=jax.ShapeDtypeStruct((_N, _D), jnp.float32),
)


def kernel(x, edge_index, W1, b1, W2, b2):
    src = edge_index[0].reshape(_NW, _NCH, _C)
    dst = edge_index[1].reshape(_NW, _NCH, _C)
    deg = _deg_sc(dst).reshape(_NC, _NR)
    deg_nt = deg.T
    xs_lo, xs_hi = _scale_tc(deg_nt, x)
    g_lo = _prop_sc(xs_lo, src, dst)
    g_hi = _prop_sc(xs_hi, src, dst)
    h2s_lo, h2s_hi = _mid_tc(deg_nt, g_lo, g_hi, xs_lo, xs_hi,
                             W1, b1.reshape(1, _DH), W2)
    q_lo = _prop_sc(h2s_lo, src, dst)
    q_hi = _prop_sc(h2s_hi, src, dst)
    return _final_tc(deg_nt, q_lo, q_hi, h2s_lo, h2s_hi, b2.reshape(1, _D))


# speedup vs baseline: 44.8413x; 1.3599x over previous
import functools

import jax
import jax.numpy as jnp
from jax import lax
from jax.experimental import pallas as pl
from jax.experimental.pallas import tpu as pltpu
from jax.experimental.pallas import tpu_sc as plsc

_N = 10000
_D = 128
_DH = 256
_DHALF = _D // 2

_NC = 2
_NS = 16
_NW = _NC * _NS
_NCH = 125
_C = 80
_NB = 5
_RPS = 640
_NR = _NS * _RPS

_mesh = plsc.VectorSubcoreMesh(core_axis_name="c", subcore_axis_name="s")


@functools.partial(
    pl.kernel,
    out_type=jax.ShapeDtypeStruct((_NC * _NR,), jnp.float32),
    mesh=_mesh,
    scratch_types=[
        pltpu.VMEM((_NCH, _C), jnp.int32),
        pltpu.VMEM((_C,), jnp.float32),
        pltpu.VMEM((_RPS,), jnp.float32),
        pltpu.VMEM_SHARED((_NR,), jnp.float32),
    ],
)
def _deg_sc(dst_hbm, out_hbm, dst_v, ones_v, z_v, acc_sh):
    c = lax.axis_index("c")
    s = lax.axis_index("s")
    wid = c * _NS + s
    pltpu.sync_copy(dst_hbm.at[wid], dst_v)
    for i in range(_C // 16):
        ones_v[pl.ds(i * 16, 16)] = jnp.ones((16,), jnp.float32)
    for i in range(_RPS // 16):
        z_v[pl.ds(i * 16, 16)] = jnp.zeros((16,), jnp.float32)
    pltpu.sync_copy(z_v, acc_sh.at[pl.ds(s * _RPS, _RPS)])
    plsc.subcore_barrier()

    def body(j, carry):
        pltpu.sync_copy(ones_v.at[pl.ds(0, _C)], acc_sh.at[dst_v.at[j]], add=True)
        return carry

    lax.fori_loop(0, _NCH, body, 0)
    plsc.subcore_barrier()
    pltpu.sync_copy(acc_sh.at[pl.ds(s * _RPS, _RPS)],
                    out_hbm.at[pl.ds(c * _NR + s * _RPS, _RPS)])


@functools.partial(
    pl.kernel,
    out_type=jax.ShapeDtypeStruct((_NC, _NR, _D), jnp.bfloat16),
    mesh=_mesh,
    scratch_types=[
        pltpu.VMEM((_NCH, _C), jnp.int32),
        pltpu.VMEM((_NCH, _C), jnp.int32),
        pltpu.VMEM((_NB, _C, _D), jnp.bfloat16),
        pltpu.VMEM_SHARED((_NR, _D), jnp.bfloat16),
    ] + [pltpu.SemaphoreType.DMA] * _NB,
    compiler_params=pltpu.CompilerParams(use_tc_tiling_on_sc=False),
)
def _prop_sc(tab, src_hbm, dst_hbm, out_hbm, src_v, dst_v, rows_v, acc_sh, *sems):
    c = lax.axis_index("c")
    s = lax.axis_index("s")
    wid = c * _NS + s
    pltpu.sync_copy(src_hbm.at[wid], src_v)
    pltpu.sync_copy(dst_hbm.at[wid], dst_v)
    rs = s * _RPS
    pltpu.sync_copy(tab.at[pl.ds(rs, _RPS)], acc_sh.at[pl.ds(rs, _RPS)])
    plsc.subcore_barrier()

    for b in range(_NB):
        pltpu.async_copy(tab.at[src_v.at[b]], rows_v.at[b], sems[b])

    def round_body(r, carry):
        for b in range(_NB):
            g = r * _NB + b
            pltpu.make_async_copy(tab.at[src_v.at[g]], rows_v.at[b],
                                  sems[b]).wait()
            pltpu.sync_copy(rows_v.at[b], acc_sh.at[dst_v.at[g]], add=True)

            @pl.when(r < _NCH // _NB - 1)
            def _():
                pltpu.async_copy(tab.at[src_v.at[g + _NB]], rows_v.at[b],
                                 sems[b])
        return carry

    lax.fori_loop(0, _NCH // _NB, round_body, 0)
    plsc.subcore_barrier()
    pltpu.sync_copy(acc_sh.at[pl.ds(rs, _RPS)], out_hbm.at[c, pl.ds(rs, _RPS)])


_BN = 2000
_GRID = _N // _BN


def _dinv_of(deg_ref):
    i = pl.program_id(0)
    blk = deg_ref[pl.ds(i * _BN, _BN), :]
    d = blk[:, 0:1] + blk[:, 1:2] + 1.0
    return lax.rsqrt(d)


def _scale_body(deg_ref, x_ref, o_ref):
    dinv = _dinv_of(deg_ref)
    o_ref[...] = (x_ref[...] * dinv).astype(jnp.bfloat16)


def _mid_body(deg_ref, g_ref, xs_ref, w1_ref, b1_ref, w2_ref, o_ref):
    dinv = _dinv_of(deg_ref)
    gsum = (g_ref[0].astype(jnp.float32) + g_ref[1].astype(jnp.float32)
            - xs_ref[...].astype(jnp.float32))
    p = gsum * dinv
    h = jnp.maximum(
        jnp.dot(p, w1_ref[...], preferred_element_type=jnp.float32)
        + b1_ref[...], 0.0)
    h2 = jnp.dot(h, w2_ref[...], preferred_element_type=jnp.float32)
    o_ref[...] = (h2 * dinv).astype(jnp.bfloat16)


def _final_body(deg_ref, q_ref, hs_ref, b2_ref, o_ref):
    dinv = _dinv_of(deg_ref)
    qsum = (q_ref[0].astype(jnp.float32) + q_ref[1].astype(jnp.float32)
            - hs_ref[...].astype(jnp.float32))
    o = qsum * dinv + b2_ref[...]
    m = jnp.max(o, axis=1, keepdims=True)
    lse = jnp.log(jnp.sum(jnp.exp(o - m), axis=1, keepdims=True))
    o_ref[...] = o - m - lse


_deg_spec = pl.BlockSpec((_NR, _NC), lambda i: (0, 0))
_row_spec = pl.BlockSpec((_BN, _D), lambda i: (i, 0))
_par_spec = pl.BlockSpec((_NC, _BN, _D), lambda i: (0, i, 0))
_tab_out = jax.ShapeDtypeStruct((_NR, _D), jnp.bfloat16)

_scale_tc = pl.pallas_call(
    _scale_body,
    grid=(_GRID,),
    in_specs=[_deg_spec, _row_spec],
    out_specs=_row_spec,
    out_shape=_tab_out,
)

_mid_tc = pl.pallas_call(
    _mid_body,
    grid=(_GRID,),
    in_specs=[
        _deg_spec,
        _par_spec,
        _row_spec,
        pl.BlockSpec((_D, _DH), lambda i: (0, 0)),
        pl.BlockSpec((1, _DH), lambda i: (0, 0)),
        pl.BlockSpec((_DH, _D), lambda i: (0, 0)),
    ],
    out_specs=_row_spec,
    out_shape=_tab_out,
)

_final_tc = pl.pallas_call(
    _final_body,
    grid=(_GRID,),
    in_specs=[_deg_spec, _par_spec, _row_spec,
              pl.BlockSpec((1, _D), lambda i: (0, 0))],
    out_specs=_row_spec,
    out_shape=jax.ShapeDtypeStruct((_N, _D), jnp.float32),
)


def kernel(x, edge_index, W1, b1, W2, b2):
    src = edge_index[0].reshape(_NW, _NCH, _C)
    dst = edge_index[1].reshape(_NW, _NCH, _C)
    deg = _deg_sc(dst).reshape(_NC, _NR)
    deg_nt = deg.T
    xs = _scale_tc(deg_nt, x)
    g = _prop_sc(xs, src, dst)
    h2s = _mid_tc(deg_nt, g, xs, W1, b1.reshape(1, _DH), W2)
    q = _prop_sc(h2s, src, dst)
    return _final_tc(deg_nt, q, h2s, b2.reshape(1, _D))


# speedup vs baseline: 45.0609x; 1.0049x over previous
import functools

import jax
import jax.numpy as jnp
from jax import lax
from jax.experimental import pallas as pl
from jax.experimental.pallas import tpu as pltpu
from jax.experimental.pallas import tpu_sc as plsc

_N = 10000
_D = 128
_DH = 256
_DHALF = _D // 2

_NC = 2
_NS = 16
_NW = _NC * _NS
_NCH = 125
_C = 80
_NB = 5
_RPS = 640
_NR = _NS * _RPS

_mesh = plsc.VectorSubcoreMesh(core_axis_name="c", subcore_axis_name="s")


@functools.partial(
    pl.kernel,
    out_type=jax.ShapeDtypeStruct((_NC * _NR,), jnp.float32),
    mesh=_mesh,
    scratch_types=[
        pltpu.VMEM((_NCH, _C), jnp.int32),
        pltpu.VMEM((_C,), jnp.float32),
        pltpu.VMEM((_RPS,), jnp.float32),
        pltpu.VMEM_SHARED((_NR,), jnp.float32),
    ],
    compiler_params=pltpu.CompilerParams(use_tc_tiling_on_sc=False),
)
def _deg_sc(dst_hbm, out_hbm, dst_v, ones_v, z_v, acc_sh):
    c = lax.axis_index("c")
    s = lax.axis_index("s")
    wid = c * _NS + s
    pltpu.sync_copy(dst_hbm.at[wid], dst_v)
    for i in range(_C // 16):
        ones_v[pl.ds(i * 16, 16)] = jnp.ones((16,), jnp.float32)
    for i in range(_RPS // 16):
        z_v[pl.ds(i * 16, 16)] = jnp.zeros((16,), jnp.float32)
    pltpu.sync_copy(z_v, acc_sh.at[pl.ds(s * _RPS, _RPS)])
    plsc.subcore_barrier()

    def body(j, carry):
        pltpu.sync_copy(ones_v.at[pl.ds(0, _C)], acc_sh.at[dst_v.at[j]], add=True)
        return carry

    lax.fori_loop(0, _NCH, body, 0)
    plsc.subcore_barrier()
    pltpu.sync_copy(acc_sh.at[pl.ds(s * _RPS, _RPS)],
                    out_hbm.at[pl.ds(c * _NR + s * _RPS, _RPS)])


@functools.partial(
    pl.kernel,
    out_type=jax.ShapeDtypeStruct((_NC, _NR, _D), jnp.bfloat16),
    mesh=_mesh,
    scratch_types=[
        pltpu.VMEM((_NCH, _C), jnp.int32),
        pltpu.VMEM((_NCH, _C), jnp.int32),
        pltpu.VMEM((_NB, _C, _D), jnp.bfloat16),
        pltpu.VMEM_SHARED((_NR, _D), jnp.bfloat16),
    ] + [pltpu.SemaphoreType.DMA] * _NB,
    compiler_params=pltpu.CompilerParams(use_tc_tiling_on_sc=False),
)
def _prop_sc(tab, src_hbm, dst_hbm, out_hbm, src_v, dst_v, rows_v, acc_sh, *sems):
    c = lax.axis_index("c")
    s = lax.axis_index("s")
    wid = c * _NS + s
    pltpu.sync_copy(src_hbm.at[wid], src_v)
    pltpu.sync_copy(dst_hbm.at[wid], dst_v)
    rs = s * _RPS
    pltpu.sync_copy(tab.at[pl.ds(rs, _RPS)], acc_sh.at[pl.ds(rs, _RPS)])
    plsc.subcore_barrier()

    for b in range(_NB):
        pltpu.async_copy(tab.at[src_v.at[b]], rows_v.at[b], sems[b])

    def round_body(r, carry):
        for b in range(_NB):
            g = r * _NB + b
            pltpu.make_async_copy(tab.at[src_v.at[g]], rows_v.at[b],
                                  sems[b]).wait()
            pltpu.sync_copy(rows_v.at[b], acc_sh.at[dst_v.at[g]], add=True)

            @pl.when(r < _NCH // _NB - 1)
            def _():
                pltpu.async_copy(tab.at[src_v.at[g + _NB]], rows_v.at[b],
                                 sems[b])
        return carry

    lax.fori_loop(0, _NCH // _NB, round_body, 0)
    plsc.subcore_barrier()
    pltpu.sync_copy(acc_sh.at[pl.ds(rs, _RPS)], out_hbm.at[c, pl.ds(rs, _RPS)])


_BN = 2000
_GRID = _N // _BN


def _dinv_of(deg_ref):
    i = pl.program_id(0)
    blk = deg_ref[pl.ds(i * _BN, _BN), :]
    d = blk[:, 0:1] + blk[:, 1:2] + 1.0
    return lax.rsqrt(d)


def _scale_body(deg_ref, x_ref, o_ref):
    dinv = _dinv_of(deg_ref)
    o_ref[...] = (x_ref[...] * dinv).astype(jnp.bfloat16)


def _mid_body(deg_ref, g_ref, xs_ref, w1_ref, b1_ref, w2_ref, o_ref):
    dinv = _dinv_of(deg_ref)
    gsum = (g_ref[0].astype(jnp.float32) + g_ref[1].astype(jnp.float32)
            - xs_ref[...].astype(jnp.float32))
    p = gsum * dinv
    h = jnp.maximum(
        jnp.dot(p, w1_ref[...], preferred_element_type=jnp.float32)
        + b1_ref[...], 0.0)
    h2 = jnp.dot(h, w2_ref[...], preferred_element_type=jnp.float32)
    o_ref[...] = (h2 * dinv).astype(jnp.bfloat16)


def _final_body(deg_ref, q_ref, hs_ref, b2_ref, o_ref):
    dinv = _dinv_of(deg_ref)
    qsum = (q_ref[0].astype(jnp.float32) + q_ref[1].astype(jnp.float32)
            - hs_ref[...].astype(jnp.float32))
    o = qsum * dinv + b2_ref[...]
    m = jnp.max(o, axis=1, keepdims=True)
    lse = jnp.log(jnp.sum(jnp.exp(o - m), axis=1, keepdims=True))
    o_ref[...] = o - m - lse


_deg_spec = pl.BlockSpec((_NR, _NC), lambda i: (0, 0))
_row_spec = pl.BlockSpec((_BN, _D), lambda i: (i, 0))
_par_spec = pl.BlockSpec((_NC, _BN, _D), lambda i: (0, i, 0))
_tab_out = jax.ShapeDtypeStruct((_NR, _D), jnp.bfloat16)

_scale_tc = pl.pallas_call(
    _scale_body,
    grid=(_GRID,),
    in_specs=[_deg_spec, _row_spec],
    out_specs=_row_spec,
    out_shape=_tab_out,
)

_mid_tc = pl.pallas_call(
    _mid_body,
    grid=(_GRID,),
    in_specs=[
        _deg_spec,
        _par_spec,
        _row_spec,
        pl.BlockSpec((_D, _DH), lambda i: (0, 0)),
        pl.BlockSpec((1, _DH), lambda i: (0, 0)),
        pl.BlockSpec((_DH, _D), lambda i: (0, 0)),
    ],
    out_specs=_row_spec,
    out_shape=_tab_out,
)

_final_tc = pl.pallas_call(
    _final_body,
    grid=(_GRID,),
    in_specs=[_deg_spec, _par_spec, _row_spec,
              pl.BlockSpec((1, _D), lambda i: (0, 0))],
    out_specs=_row_spec,
    out_shape=jax.ShapeDtypeStruct((_N, _D), jnp.float32),
)


def kernel(x, edge_index, W1, b1, W2, b2):
    src = edge_index[0].reshape(_NW, _NCH, _C)
    dst = edge_index[1].reshape(_NW, _NCH, _C)
    deg = _deg_sc(dst).reshape(_NC, _NR)
    deg_nt = deg.T
    xs = _scale_tc(deg_nt, x)
    g = _prop_sc(xs, src, dst)
    h2s = _mid_tc(deg_nt, g, xs, W1, b1.reshape(1, _DH), W2)
    q = _prop_sc(h2s, src, dst)
    return _final_tc(deg_nt, q, h2s, b2.reshape(1, _D))
